# baseline (speedup 1.0000x reference)
import jax
import jax.numpy as jnp
from jax import lax
from jax.experimental import pallas as pl
from jax.experimental.pallas import tpu as pltpu
from jax.experimental.pallas import tpu_sc as plsc

N = 10000
E = 320000
F = 128

NC = 2
NS = 16
L = 16

FPT = F // NS
EPC = E // NC
CHUNK = 800
NCHUNK = EPC // CHUNK
WAVES = CHUNK // L
NH = N // 2


def _spmm_entry(tbl_hbm, src_hbm, dst_hbm, w_hbm, parts_hbm,
                table, acc0, acc1, src_a, dst_a, w_a, src_b, dst_b, w_b,
                tsem_a, tsem_b):
    c = lax.axis_index("c")
    s = lax.axis_index("s")

    def tstart(ci, sb, db, wb, sem):
        off = c * EPC + ci * CHUNK
        pltpu.async_copy(src_hbm.at[pl.ds(off, CHUNK)], sb, sem)
        pltpu.async_copy(dst_hbm.at[pl.ds(off, CHUNK)], db, sem)
        pltpu.async_copy(w_hbm.at[pl.ds(off, CHUNK)], wb, sem)

    def twait(sb, db, wb, sem):
        pltpu.make_async_copy(src_hbm.at[pl.ds(0, CHUNK)], sb, sem).wait()
        pltpu.make_async_copy(dst_hbm.at[pl.ds(0, CHUNK)], db, sem).wait()
        pltpu.make_async_copy(w_hbm.at[pl.ds(0, CHUNK)], wb, sem).wait()

    def process(sb, db, wb):
        def wave(t, carry):
            o = t * L
            src16 = sb[pl.ds(o, L)]
            dst16 = db[pl.ds(o, L)]
            w16 = wb[pl.ds(o, L)]
            for p in range(FPT // 2):
                pidx = jnp.full((L,), p, jnp.int32)
                v = plsc.load_gather(table, [pidx, src16])
                lo = plsc.bitcast(lax.shift_left(v, 16), jnp.float32)
                hi = plsc.bitcast(v & jnp.int32(-65536), jnp.float32)
                pidx32 = jnp.full((L,), p, jnp.int32)
                plsc.addupdate_scatter(acc0, [pidx32, dst16], lo * w16)
                plsc.addupdate_scatter(acc1, [pidx32, dst16], hi * w16)
            return carry
        lax.fori_loop(0, WAVES, wave, 0, unroll=4)

    pltpu.sync_copy(tbl_hbm.at[pl.ds(FPT // 2 * s, FPT // 2)], table)
    zeros = jnp.zeros((L,), jnp.float32)

    def zbody(i, carry):
        for f in range(FPT // 2):
            acc0[f, pl.ds(i * L, L)] = zeros
            acc1[f, pl.ds(i * L, L)] = zeros
        return carry
    lax.fori_loop(0, N // L, zbody, 0)

    tstart(0, src_a, dst_a, w_a, tsem_a)
    tstart(1, src_b, dst_b, w_b, tsem_b)

    def group(gi, carry):
        ci = 2 * gi
        twait(src_a, dst_a, w_a, tsem_a)
        process(src_a, dst_a, w_a)

        @pl.when(ci + 2 < NCHUNK)
        def _():
            tstart(ci + 2, src_a, dst_a, w_a, tsem_a)

        twait(src_b, dst_b, w_b, tsem_b)
        process(src_b, dst_b, w_b)

        @pl.when(ci + 3 < NCHUNK)
        def _():
            tstart(ci + 3, src_b, dst_b, w_b, tsem_b)

        return carry

    lax.fori_loop(0, NCHUNK // 2, group, 0)

    for p in range(FPT // 2):
        pltpu.sync_copy(acc0.at[p], parts_hbm.at[c, FPT * s + 2 * p])
        pltpu.sync_copy(acc1.at[p], parts_hbm.at[c, FPT * s + 2 * p + 1])


_spmm = pl.kernel(
    _spmm_entry,
    out_type=jax.ShapeDtypeStruct((NC, F, N), jnp.float32),
    mesh=plsc.VectorSubcoreMesh(core_axis_name="c", subcore_axis_name="s"),
    compiler_params=pltpu.CompilerParams(needs_layout_passes=False),
    scratch_types=[
        pltpu.VMEM((FPT // 2, N), jnp.int32),
        pltpu.VMEM((FPT // 2, N), jnp.float32),
        pltpu.VMEM((FPT // 2, N), jnp.float32),
        pltpu.VMEM((CHUNK,), jnp.int32),
        pltpu.VMEM((CHUNK,), jnp.int32),
        pltpu.VMEM((CHUNK,), jnp.float32),
        pltpu.VMEM((CHUNK,), jnp.int32),
        pltpu.VMEM((CHUNK,), jnp.int32),
        pltpu.VMEM((CHUNK,), jnp.float32),
        pltpu.SemaphoreType.DMA,
        pltpu.SemaphoreType.DMA,
    ],
)


def _mmT_kernel(x_ref, w_ref, o_ref):
    o_ref[...] = lax.dot_general(
        w_ref[...], x_ref[...], (((0,), (1,)), ((), ())),
        preferred_element_type=jnp.float32).astype(jnp.bfloat16)


def _midT_kernel(p_ref, b_ref, w_ref, o_ref):
    h = jnp.maximum(p_ref[0] + p_ref[1] + b_ref[...], 0.0)
    o_ref[...] = lax.dot_general(
        w_ref[...], h, (((0,), (0,)), ((), ())),
        preferred_element_type=jnp.float32).astype(jnp.bfloat16)


def _outT_kernel(p_ref, b_ref, o_ref):
    o_ref[...] = jnp.maximum(p_ref[0] + p_ref[1] + b_ref[...], 0.0).T


def _mmT(x, w):
    return pl.pallas_call(
        _mmT_kernel,
        out_shape=jax.ShapeDtypeStruct((F, N), jnp.bfloat16),
    )(x, w)


def _midT(parts, b, w):
    return pl.pallas_call(
        _midT_kernel,
        out_shape=jax.ShapeDtypeStruct((F, N), jnp.bfloat16),
    )(parts, b.reshape(F, 1), w)


def _finalT(parts, b):
    return pl.pallas_call(
        _outT_kernel,
        out_shape=jax.ShapeDtypeStruct((N, F), jnp.float32),
    )(parts, b.reshape(F, 1))


def _pack(sT):
    return lax.bitcast_convert_type(
        jnp.transpose(sT.reshape(F // 2, 2, N), (0, 2, 1)), jnp.int32)


def kernel(x, edge_index, edge_weight, W1, b1, W2, b2):
    src = edge_index[0]
    dst = edge_index[1]
    wgt = edge_weight
    s1 = _mmT(x, W1)
    p1 = _spmm(_pack(s1), src, dst, wgt)
    s2 = _midT(p1, b1, W2)
    p2 = _spmm(_pack(s2), src, dst, wgt)
    return _finalT(p2, b2)


# speedup vs baseline: 1.1608x; 1.1608x over previous
import jax
import jax.numpy as jnp
from jax import lax
from jax.experimental import pallas as pl
from jax.experimental.pallas import tpu as pltpu
from jax.experimental.pallas import tpu_sc as plsc

N = 10000
E = 320000
F = 128

NC = 2
NS = 16
NW = NC * NS
L = 16

EPW = E // NW
C = 80
NCHUNK = EPW // C
N_PAD = 10240
RPT = N_PAD // NS


def _spmm_entry(sup_hbm, src_hbm, dst_hbm, w_hbm, zero_hbm, parts_hbm,
                src_v, w_v, dst_c, rows_v, acc, gsem):
    c = lax.axis_index("c")
    s = lax.axis_index("s")
    wid = c * NS + s
    base = wid * EPW

    pltpu.sync_copy(zero_hbm.at[pl.ds(s * RPT, RPT)],
                    acc.at[pl.ds(s * RPT, RPT)])
    pltpu.sync_copy(src_hbm.at[pl.ds(base, EPW)], src_v)
    pltpu.sync_copy(w_hbm.at[pl.ds(base, EPW)], w_v)
    plsc.subcore_barrier()

    def chunk_body(i, carry):
        off = i * C
        pltpu.sync_copy(dst_hbm.at[pl.ds(base + off, C)], dst_c)
        pltpu.async_copy(sup_hbm.at[src_v.at[pl.ds(off, C)]], rows_v,
                         gsem).wait()

        def row_body(r, rcarry):
            wb = plsc.load_gather(w_v, [lax.broadcast(off + r, (L,))])
            for j in range(F // L):
                sl = (r, pl.ds(j * L, L))
                rows_v[sl] = rows_v[sl] * wb
            return rcarry

        lax.fori_loop(0, C, row_body, 0)
        pltpu.sync_copy(rows_v, acc.at[dst_c], add=True)
        return carry

    lax.fori_loop(0, NCHUNK, chunk_body, 0)
    plsc.subcore_barrier()
    pltpu.sync_copy(acc.at[pl.ds(s * RPT, RPT)],
                    parts_hbm.at[c, pl.ds(s * RPT, RPT)])


_spmm = pl.kernel(
    _spmm_entry,
    out_type=jax.ShapeDtypeStruct((NC, N_PAD, F), jnp.float32),
    mesh=plsc.VectorSubcoreMesh(core_axis_name="c", subcore_axis_name="s"),
    compiler_params=pltpu.CompilerParams(needs_layout_passes=False),
    scratch_types=[
        pltpu.VMEM((EPW,), jnp.int32),
        pltpu.VMEM((EPW,), jnp.float32),
        pltpu.VMEM((C,), jnp.int32),
        pltpu.VMEM((C, F), jnp.float32),
        pltpu.VMEM_SHARED((N_PAD, F), jnp.float32),
        pltpu.SemaphoreType.DMA,
    ],
)


def _mm_kernel(x_ref, w_ref, o_ref):
    o_ref[...] = jnp.dot(x_ref[...], w_ref[...],
                         preferred_element_type=jnp.float32)


def _mid_kernel(p_ref, b_ref, w_ref, o_ref):
    h = jnp.maximum(p_ref[0] + p_ref[1] + b_ref[...], 0.0)
    o_ref[...] = jnp.dot(h, w_ref[...], preferred_element_type=jnp.float32)


def _out_kernel(p_ref, b_ref, o_ref):
    o_ref[...] = jnp.maximum(p_ref[0] + p_ref[1] + b_ref[...], 0.0)


_BM = 2000


def _mm(x, w):
    return pl.pallas_call(
        _mm_kernel,
        grid=(N // _BM,),
        in_specs=[pl.BlockSpec((_BM, F), lambda i: (i, 0)),
                  pl.BlockSpec((F, F), lambda i: (0, 0))],
        out_specs=pl.BlockSpec((_BM, F), lambda i: (i, 0)),
        out_shape=jax.ShapeDtypeStruct((N, F), jnp.float32),
    )(x, w)


def _mid(parts, b, w):
    return pl.pallas_call(
        _mid_kernel,
        grid=(N // _BM,),
        in_specs=[pl.BlockSpec((NC, _BM, F), lambda i: (0, i, 0)),
                  pl.BlockSpec((1, F), lambda i: (0, 0)),
                  pl.BlockSpec((F, F), lambda i: (0, 0))],
        out_specs=pl.BlockSpec((_BM, F), lambda i: (i, 0)),
        out_shape=jax.ShapeDtypeStruct((N, F), jnp.float32),
    )(parts, b.reshape(1, F), w)


def _final(parts, b):
    return pl.pallas_call(
        _out_kernel,
        grid=(N // _BM,),
        in_specs=[pl.BlockSpec((NC, _BM, F), lambda i: (0, i, 0)),
                  pl.BlockSpec((1, F), lambda i: (0, 0))],
        out_specs=pl.BlockSpec((_BM, F), lambda i: (i, 0)),
        out_shape=jax.ShapeDtypeStruct((N, F), jnp.float32),
    )(parts, b.reshape(1, F))


def kernel(x, edge_index, edge_weight, W1, b1, W2, b2):
    src = edge_index[0]
    dst = edge_index[1]
    zeros = jnp.zeros((N_PAD, F), jnp.float32)
    s1 = _mm(x, W1)
    parts1 = _spmm(s1, src, dst, edge_weight, zeros)
    s2 = _mid(parts1, b1, W2)
    parts2 = _spmm(s2, src, dst, edge_weight, zeros)
    return _final(parts2, b2)


# speedup vs baseline: 1.3412x; 1.1554x over previous
import jax
import jax.numpy as jnp
from jax import lax
from jax.experimental import pallas as pl
from jax.experimental.pallas import tpu as pltpu
from jax.experimental.pallas import tpu_sc as plsc

N = 10000
E = 320000
F = 128

NC = 2
NS = 16
NW = NC * NS
L = 16

EPW = E // NW
C = 80
NCHUNK = EPW // C
N_PAD = 10240
RPT = N_PAD // NS


def _spmm_entry(sup_hbm, src_hbm, dst_hbm, w_hbm, zero_hbm, parts_hbm,
                src_v, w_v, dst_c, rows_v, acc, gsem):
    c = lax.axis_index("c")
    s = lax.axis_index("s")
    wid = c * NS + s
    base = wid * EPW

    pltpu.sync_copy(zero_hbm.at[pl.ds(s * RPT, RPT)],
                    acc.at[pl.ds(s * RPT, RPT)])
    pltpu.sync_copy(src_hbm.at[pl.ds(base, EPW)], src_v)
    pltpu.sync_copy(w_hbm.at[pl.ds(base, EPW)], w_v)
    plsc.subcore_barrier()

    def chunk_body(i, carry):
        off = i * C
        cp = pltpu.async_copy(sup_hbm.at[src_v.at[pl.ds(off, C)]], rows_v,
                              gsem)
        pltpu.sync_copy(dst_hbm.at[pl.ds(base + off, C)], dst_c)
        cp.wait()

        def row_body(r, rcarry):
            wb = plsc.load_gather(w_v, [lax.broadcast(off + r, (L,))])
            for j in range(F // L):
                sl = (r, pl.ds(j * L, L))
                rows_v[sl] = rows_v[sl] * wb
            return rcarry

        lax.fori_loop(0, C, row_body, 0)
        pltpu.sync_copy(rows_v, acc.at[dst_c], add=True)
        return carry

    lax.fori_loop(0, NCHUNK, chunk_body, 0)
    plsc.subcore_barrier()
    pltpu.sync_copy(acc.at[pl.ds(s * RPT, RPT)],
                    parts_hbm.at[c, pl.ds(s * RPT, RPT)])


_spmm = pl.kernel(
    _spmm_entry,
    out_type=jax.ShapeDtypeStruct((NC, N_PAD, F), jnp.float32),
    mesh=plsc.VectorSubcoreMesh(core_axis_name="c", subcore_axis_name="s"),
    compiler_params=pltpu.CompilerParams(needs_layout_passes=False),
    scratch_types=[
        pltpu.VMEM((EPW,), jnp.int32),
        pltpu.VMEM((EPW,), jnp.float32),
        pltpu.VMEM((C,), jnp.int32),
        pltpu.VMEM((C, F), jnp.float32),
        pltpu.VMEM_SHARED((N_PAD, F), jnp.float32),
        pltpu.SemaphoreType.DMA,
    ],
)


def _mm_kernel(x_ref, w_ref, o_ref):
    o_ref[...] = jnp.dot(x_ref[...], w_ref[...],
                         preferred_element_type=jnp.float32)


def _mid_kernel(p_ref, b_ref, w_ref, o_ref):
    h = jnp.maximum(p_ref[0] + p_ref[1] + b_ref[...], 0.0)
    o_ref[...] = jnp.dot(h, w_ref[...], preferred_element_type=jnp.float32)


def _out_kernel(p_ref, b_ref, o_ref):
    o_ref[...] = jnp.maximum(p_ref[0] + p_ref[1] + b_ref[...], 0.0)


_BM = 2000


def _mm(x, w):
    return pl.pallas_call(
        _mm_kernel,
        grid=(N // _BM,),
        in_specs=[pl.BlockSpec((_BM, F), lambda i: (i, 0)),
                  pl.BlockSpec((F, F), lambda i: (0, 0))],
        out_specs=pl.BlockSpec((_BM, F), lambda i: (i, 0)),
        out_shape=jax.ShapeDtypeStruct((N, F), jnp.float32),
    )(x, w)


def _mid(parts, b, w):
    return pl.pallas_call(
        _mid_kernel,
        grid=(N // _BM,),
        in_specs=[pl.BlockSpec((NC, _BM, F), lambda i: (0, i, 0)),
                  pl.BlockSpec((1, F), lambda i: (0, 0)),
                  pl.BlockSpec((F, F), lambda i: (0, 0))],
        out_specs=pl.BlockSpec((_BM, F), lambda i: (i, 0)),
        out_shape=jax.ShapeDtypeStruct((N, F), jnp.float32),
    )(parts, b.reshape(1, F), w)


def _final(parts, b):
    return pl.pallas_call(
        _out_kernel,
        grid=(N // _BM,),
        in_specs=[pl.BlockSpec((NC, _BM, F), lambda i: (0, i, 0)),
                  pl.BlockSpec((1, F), lambda i: (0, 0))],
        out_specs=pl.BlockSpec((_BM, F), lambda i: (i, 0)),
        out_shape=jax.ShapeDtypeStruct((N, F), jnp.float32),
    )(parts, b.reshape(1, F))


def kernel(x, edge_index, edge_weight, W1, b1, W2, b2):
    src = edge_index[0]
    dst = edge_index[1]
    zeros = jnp.zeros((N_PAD, F), jnp.float32)
    s1 = _mm(x, W1)
    parts1 = _spmm(s1, src, dst, edge_weight, zeros)
    s2 = _mid(parts1, b1, W2)
    parts2 = _spmm(s2, src, dst, edge_weight, zeros)
    return _final(parts2, b2)
